# baseline (speedup 1.0000x reference)
import functools

import jax
import jax.numpy as jnp
from jax import lax
from jax.experimental import pallas as pl
from jax.experimental.pallas import tpu as pltpu
from jax.experimental.pallas import tpu_sc as plsc

NUM_ATOMIC = 119
DIM = 128
NC, NS = 2, 16
NW = NC * NS

CHUNK = 256
KIDX = CHUNK // 128


@functools.partial(jax.jit, static_argnames=("b_total",))
def _sc_embed(idx2d, table, b_total):
    b_per_w = b_total // NW
    n_chunks = b_per_w // CHUNK
    n_pairs = n_chunks // 2
    rows_per_w = b_per_w // 128

    mesh = plsc.VectorSubcoreMesh(core_axis_name="c", subcore_axis_name="s")

    @functools.partial(
        pl.kernel,
        mesh=mesh,
        out_type=jax.ShapeDtypeStruct((b_total, DIM), jnp.float32),
        scratch_types=[
            pltpu.VMEM((2, KIDX, 128), jnp.int32),
            pltpu.VMEM((2, CHUNK, DIM), jnp.float32),
            pltpu.SemaphoreType.DMA,
            pltpu.SemaphoreType.DMA,
            pltpu.SemaphoreType.DMA,
            pltpu.SemaphoreType.DMA,
        ],
    )
    def k(idx_hbm, table_hbm, out_hbm, idx_v, rows_v, g0, g1, s0, s1):
        wid = lax.axis_index("s") * NC + lax.axis_index("c")
        row_base = wid * rows_per_w
        tok_base = wid * b_per_w
        gsem = (g0, g1)
        ssem = (s0, s1)

        def chunk_of(p, b):
            return p * 2 + b

        def start_gather(g, b):
            pltpu.sync_copy(
                idx_hbm.at[pl.ds(row_base + g * KIDX, KIDX)], idx_v.at[b]
            )
            for j in range(KIDX):
                pltpu.async_copy(
                    table_hbm.at[idx_v.at[b, j]],
                    rows_v.at[b, pl.ds(j * 128, 128)],
                    gsem[b],
                )

        def wait_gather(b):
            for j in range(KIDX):
                pltpu.make_async_copy(
                    table_hbm.at[idx_v.at[b, j]],
                    rows_v.at[b, pl.ds(j * 128, 128)],
                    gsem[b],
                ).wait()

        def start_scatter(g, b):
            pltpu.async_copy(
                rows_v.at[b], out_hbm.at[pl.ds(tok_base + g * CHUNK, CHUNK)], ssem[b]
            )

        def wait_scatter(g, b):
            pltpu.make_async_copy(
                rows_v.at[b], out_hbm.at[pl.ds(tok_base + g * CHUNK, CHUNK)], ssem[b]
            ).wait()

        start_gather(0, 0)
        start_gather(1, 1)

        def body(p, carry):
            for b in range(2):
                g = chunk_of(p, b)
                wait_gather(b)
                start_scatter(g, b)
                nxt = g + 2

                @pl.when(nxt < n_chunks)
                def _():
                    wait_scatter(g, b)
                    start_gather(nxt, b)

            return carry

        lax.fori_loop(0, n_pairs, body, 0)
        wait_scatter(n_chunks - 2, 0)
        wait_scatter(n_chunks - 1, 1)

    return k(idx2d, table)


def kernel(tokens, table):
    b, t = tokens.shape
    b_total = b * t
    idx2d = tokens.reshape(b_total // 128, 128).astype(jnp.int32)
    out = _sc_embed(idx2d, table, b_total)
    return out.reshape(b, t, DIM)


# speedup vs baseline: 6.1157x; 6.1157x over previous
import functools

import jax
import jax.numpy as jnp
from jax import lax
from jax.experimental import pallas as pl
from jax.experimental.pallas import tpu as pltpu
from jax.experimental.pallas import tpu_sc as plsc

NUM_ATOMIC = 119
DIM = 128
NC, NS = 2, 16
NW = NC * NS

CHUNK = 128
NBUF = 4


@functools.partial(jax.jit, static_argnames=("b_total",))
def _sc_embed(idx2d, table, b_total):
    b_per_w = b_total // NW
    n_chunks = b_per_w // CHUNK
    rows_per_w = b_per_w // 128

    mesh = plsc.VectorSubcoreMesh(core_axis_name="c", subcore_axis_name="s")

    @functools.partial(
        pl.kernel,
        mesh=mesh,
        out_type=jax.ShapeDtypeStruct((b_total, DIM), jnp.float32),
        scratch_types=[
            pltpu.VMEM((NBUF, 1, 128), jnp.int32),
            pltpu.VMEM((NBUF, CHUNK, DIM), jnp.float32),
            pltpu.VMEM_SHARED((NUM_ATOMIC, DIM), jnp.float32),
            [pltpu.SemaphoreType.DMA] * NBUF,
            [pltpu.SemaphoreType.DMA] * NBUF,
        ],
    )
    def k(idx_hbm, table_hbm, out_hbm, idx_v, rows_v, table_sp, gsem, ssem):
        wid = lax.axis_index("s") * NC + lax.axis_index("c")
        row_base = wid * rows_per_w
        tok_base = wid * b_per_w

        @pl.when(lax.axis_index("s") == 0)
        def _():
            pltpu.sync_copy(table_hbm, table_sp)

        plsc.subcore_barrier()

        def start_gather(g, b):
            pltpu.sync_copy(idx_hbm.at[pl.ds(row_base + g, 1)], idx_v.at[b])
            pltpu.async_copy(
                table_sp.at[idx_v.at[b, 0]], rows_v.at[b], gsem[b]
            )

        def wait_gather(b):
            pltpu.make_async_copy(
                table_sp.at[idx_v.at[b, 0]], rows_v.at[b], gsem[b]
            ).wait()

        def start_scatter(g, b):
            pltpu.async_copy(
                rows_v.at[b], out_hbm.at[pl.ds(tok_base + g * CHUNK, CHUNK)], ssem[b]
            )

        def wait_scatter(g, b):
            pltpu.make_async_copy(
                rows_v.at[b], out_hbm.at[pl.ds(tok_base + g * CHUNK, CHUNK)], ssem[b]
            ).wait()

        start_gather(0, 0)
        start_gather(1, 1)

        def body(q, carry):
            for b in range(NBUF):
                g = q * NBUF + b
                wait_gather(b)
                start_scatter(g, b)
                nxt = g + 2
                bn = (b + 2) % NBUF

                @pl.when(nxt < n_chunks)
                def _():
                    @pl.when(nxt >= NBUF)
                    def _():
                        wait_scatter(nxt - NBUF, bn)

                    start_gather(nxt, bn)

            return carry

        lax.fori_loop(0, n_chunks // NBUF, body, 0)
        for b in range(NBUF):
            wait_scatter(n_chunks - NBUF + b, b)

    return k(idx2d, table)


def kernel(tokens, table):
    b, t = tokens.shape
    b_total = b * t
    idx2d = tokens.reshape(b_total // 128, 128).astype(jnp.int32)
    out = _sc_embed(idx2d, table, b_total)
    return out.reshape(b, t, DIM)


# speedup vs baseline: 6.3263x; 1.0344x over previous
import functools

import jax
import jax.numpy as jnp
from jax import lax
from jax.experimental import pallas as pl
from jax.experimental.pallas import tpu as pltpu
from jax.experimental.pallas import tpu_sc as plsc

NUM_ATOMIC = 119
DIM = 128
NC, NS = 2, 16
NW = NC * NS

CHUNK = 128
NBUF = 4
SB = 80


@functools.partial(jax.jit, static_argnames=("b_total",))
def _sc_embed(idx2d, table, b_total):
    b_per_w = b_total // NW
    n_chunks = b_per_w // CHUNK
    n_sb = n_chunks // SB
    rows_per_w = b_per_w // 128

    mesh = plsc.VectorSubcoreMesh(core_axis_name="c", subcore_axis_name="s")

    @functools.partial(
        pl.kernel,
        mesh=mesh,
        out_type=jax.ShapeDtypeStruct((b_total, DIM), jnp.float32),
        scratch_types=[
            pltpu.VMEM((2, SB, 128), jnp.int32),
            pltpu.VMEM((NBUF, CHUNK, DIM), jnp.float32),
            pltpu.VMEM_SHARED((NUM_ATOMIC, DIM), jnp.float32),
            [pltpu.SemaphoreType.DMA] * NBUF,
            pltpu.SemaphoreType.DMA,
        ],
    )
    def k(idx_hbm, table_hbm, out_hbm, idx_sb, rows_v, table_sp, ssem, isem):
        wid = lax.axis_index("s") * NC + lax.axis_index("c")
        row_base = wid * rows_per_w
        tok_base = wid * b_per_w

        @pl.when(lax.axis_index("s") == 0)
        def _():
            pltpu.sync_copy(table_hbm, table_sp)

        plsc.subcore_barrier()

        def idx_copy(sb, sbuf):
            return pltpu.make_async_copy(
                idx_hbm.at[pl.ds(row_base + sb * SB, SB)], idx_sb.at[sbuf], isem
            )

        def gather_copy(c, b, sbuf):
            return pltpu.make_async_copy(
                table_sp.at[idx_sb.at[sbuf, c]], rows_v.at[b], ssem[b]
            )

        def scatter_copy(g, b):
            return pltpu.make_async_copy(
                rows_v.at[b], out_hbm.at[pl.ds(tok_base + g * CHUNK, CHUNK)], ssem[b]
            )

        idx_copy(0, 0).start()
        idx_copy(0, 0).wait()

        def run_sb(sb, sbuf):
            gbase = sb * SB

            @pl.when(sb + 1 < n_sb)
            def _():
                idx_copy(sb + 1, 1 - sbuf).start()

            gather_copy(0, 0, sbuf).start()
            gather_copy(1, 1, sbuf).start()

            def body(q, carry):
                for b in range(NBUF):
                    c = q * NBUF + b
                    gather_copy(c, b, sbuf).wait()
                    scatter_copy(gbase + c, b).start()
                    nxt = c + 2
                    bn = (b + 2) % NBUF

                    @pl.when(nxt < SB)
                    def _():
                        @pl.when(nxt >= NBUF)
                        def _():
                            scatter_copy(gbase + nxt - NBUF, bn).wait()

                        gather_copy(nxt, bn, sbuf).start()

                return carry

            lax.fori_loop(0, SB // NBUF, body, 0)
            for b in range(NBUF):
                scatter_copy(gbase + SB - NBUF + b, b).wait()

            @pl.when(sb + 1 < n_sb)
            def _():
                idx_copy(sb + 1, 1 - sbuf).wait()

        def outer(sp, carry):
            run_sb(sp * 2, 0)
            run_sb(sp * 2 + 1, 1)
            return carry

        lax.fori_loop(0, n_sb // 2, outer, 0)

    return k(idx2d, table)


def kernel(tokens, table):
    b, t = tokens.shape
    b_total = b * t
    idx2d = tokens.reshape(b_total // 128, 128).astype(jnp.int32)
    out = _sc_embed(idx2d, table, b_total)
    return out.reshape(b, t, DIM)
